# Initial kernel scaffold; baseline (speedup 1.0000x reference)
import functools

import jax
import jax.numpy as jnp
from jax import lax
from jax.experimental import pallas as pl
from jax.experimental.pallas import tpu as pltpu
from jax.experimental.pallas import tpu_sc as plsc

N = 10000
E = 320000
D = 128
DE = 16
DO = 128

NC = 2
NS = 16
NW = NC * NS
EW = E // NW
C = 80
CITERS = EW // C
RPT = N // NS
ZR = 125


def _sc_body(nf_hbm, src_hbm, dst_hbm, ef_hbm, s1o, s2o, co,
             src_v, dst_v, rows_v, ef_v, ones_v, zb, zb2,
             s1_sh, s2_sh, c_sh, sem):
    cid = lax.axis_index("c")
    sid = lax.axis_index("s")
    wid = cid * NS + sid

    z16 = jnp.zeros((16,), jnp.float32)
    o16 = jnp.ones((16,), jnp.float32)

    def zrow(i, _):
        def zcol(j, _):
            zb[i, pl.ds(j * 16, 16)] = z16
            return 0
        lax.fori_loop(0, D // 16, zcol, 0)
        zb2[i, pl.ds(0, 16)] = z16
        return 0
    lax.fori_loop(0, ZR, zrow, 0)

    def orow(i, _):
        ones_v[i, pl.ds(0, 16)] = o16
        return 0
    lax.fori_loop(0, C, orow, 0)

    r0 = sid * RPT

    def zshared(k, _):
        off = r0 + k * ZR
        pltpu.sync_copy(zb, s1_sh.at[pl.ds(off, ZR)])
        pltpu.sync_copy(zb2, s2_sh.at[pl.ds(off, ZR)])
        pltpu.sync_copy(zb2, c_sh.at[pl.ds(off, ZR)])
        return 0
    lax.fori_loop(0, RPT // ZR, zshared, 0)
    plsc.subcore_barrier()

    e0 = wid * EW

    def step(it, _):
        base = e0 + it * C
        pltpu.sync_copy(src_hbm.at[pl.ds(base, C)], src_v)
        pltpu.sync_copy(dst_hbm.at[pl.ds(base, C)], dst_v)
        pltpu.async_copy(nf_hbm.at[src_v], rows_v, sem).wait()
        pltpu.sync_copy(ef_hbm.at[pl.ds(base, C)], ef_v)
        pltpu.sync_copy(rows_v, s1_sh.at[dst_v], add=True)
        pltpu.sync_copy(ef_v, s2_sh.at[dst_v], add=True)
        pltpu.sync_copy(ones_v, c_sh.at[dst_v], add=True)
        return 0
    lax.fori_loop(0, CITERS, step, 0)
    plsc.subcore_barrier()

    pltpu.sync_copy(s1_sh.at[pl.ds(r0, RPT)], s1o.at[cid, pl.ds(r0, RPT)])
    pltpu.sync_copy(s2_sh.at[pl.ds(r0, RPT)], s2o.at[cid, pl.ds(r0, RPT)])
    pltpu.sync_copy(c_sh.at[pl.ds(r0, RPT)], co.at[cid, pl.ds(r0, RPT)])


_sc_agg = pl.kernel(
    _sc_body,
    out_type=(
        jax.ShapeDtypeStruct((NC, N, D), jnp.float32),
        jax.ShapeDtypeStruct((NC, N, DE), jnp.float32),
        jax.ShapeDtypeStruct((NC, N, DE), jnp.float32),
    ),
    mesh=plsc.VectorSubcoreMesh(core_axis_name="c", subcore_axis_name="s"),
    scratch_types=[
        pltpu.VMEM((C,), jnp.int32),
        pltpu.VMEM((C,), jnp.int32),
        pltpu.VMEM((C, D), jnp.float32),
        pltpu.VMEM((C, DE), jnp.float32),
        pltpu.VMEM((C, DE), jnp.float32),
        pltpu.VMEM((ZR, D), jnp.float32),
        pltpu.VMEM((ZR, DE), jnp.float32),
        pltpu.VMEM_SHARED((N, D), jnp.float32),
        pltpu.VMEM_SHARED((N, DE), jnp.float32),
        pltpu.VMEM_SHARED((N, DE), jnp.float32),
        pltpu.SemaphoreType.DMA,
    ],
)

BN = 1000


def _tc_body(nf, s1p, s2p, cp, w1t, w2t, wa1t, wa2t, bm, ba, out):
    s1 = s1p[0] + s1p[1]
    s2 = s2p[0] + s2p[1]
    cnt = (cp[0] + cp[1])[:, 0:1]
    msum = (jnp.dot(s1, w1t[...], preferred_element_type=jnp.float32)
            + jnp.dot(s2, w2t[...], preferred_element_type=jnp.float32)
            + cnt * bm[...])
    hn = msum / jnp.maximum(cnt, 1.0)
    h = (jnp.dot(nf[...], wa1t[...], preferred_element_type=jnp.float32)
         + jnp.dot(hn, wa2t[...], preferred_element_type=jnp.float32)
         + ba[...])
    out[...] = jnp.maximum(h, 0.0)


_tc_apply = pl.pallas_call(
    _tc_body,
    grid=(N // BN,),
    in_specs=[
        pl.BlockSpec((BN, D), lambda i: (i, 0)),
        pl.BlockSpec((NC, BN, D), lambda i: (0, i, 0)),
        pl.BlockSpec((NC, BN, DE), lambda i: (0, i, 0)),
        pl.BlockSpec((NC, BN, DE), lambda i: (0, i, 0)),
        pl.BlockSpec((D, DO), lambda i: (0, 0)),
        pl.BlockSpec((DE, DO), lambda i: (0, 0)),
        pl.BlockSpec((D, DO), lambda i: (0, 0)),
        pl.BlockSpec((DO, DO), lambda i: (0, 0)),
        pl.BlockSpec((1, DO), lambda i: (0, 0)),
        pl.BlockSpec((1, DO), lambda i: (0, 0)),
    ],
    out_specs=pl.BlockSpec((BN, DO), lambda i: (i, 0)),
    out_shape=jax.ShapeDtypeStruct((N, DO), jnp.float32),
)


def kernel(nfeats, edge_index, efeats, W_msg, b_msg, W_apply, b_apply):
    src = edge_index[0]
    dst = edge_index[1]
    s1p, s2p, cp = _sc_agg(nfeats, src, dst, efeats)
    w1t = W_msg[:, :D].T
    w2t = W_msg[:, D:].T
    wa1t = W_apply[:, :D].T
    wa2t = W_apply[:, D:].T
    return _tc_apply(nfeats, s1p, s2p, cp, w1t, w2t, wa1t, wa2t,
                     b_msg.reshape(1, DO), b_apply.reshape(1, DO))


# speedup vs baseline: 2.4244x; 2.4244x over previous
import jax
import jax.numpy as jnp
from jax import lax
from jax.experimental import pallas as pl
from jax.experimental.pallas import tpu as pltpu
from jax.experimental.pallas import tpu_sc as plsc

N = 10000
E = 320000
D = 128
DE = 16
DO = 128
DA = 32

NC = 2
NS = 16
H = 5120
HA = H + 8
EPT = E // NS
C = 80
CIT = EPT // C
RPW = H // NS


def _sc_body(nf_hbm, src_hbm, dst_hbm, ef_hbm, zbig_hbm, zsmall_hbm,
             s1o, s2o,
             src_v, dst_v, rows_v, efa_v,
             s1_sh, s2_sh, sem):
    cid = lax.axis_index("c")
    sid = lax.axis_index("s")
    off = cid * H

    r0 = sid * RPW
    pltpu.sync_copy(zbig_hbm, rows_v)
    pltpu.sync_copy(zsmall_hbm, efa_v)
    for k in range(RPW // C):
        pltpu.sync_copy(rows_v, s1_sh.at[pl.ds(r0 + k * C, C)])
        pltpu.sync_copy(efa_v, s2_sh.at[pl.ds(r0 + k * C, C)])
    plsc.subcore_barrier()

    e0 = sid * EPT

    def step(it, _):
        base = e0 + it * C
        pltpu.sync_copy(src_hbm.at[pl.ds(base, C)], src_v)
        pltpu.sync_copy(dst_hbm.at[pl.ds(cid * E + base, C)], dst_v)
        pltpu.sync_copy(nf_hbm.at[src_v], rows_v)
        pltpu.sync_copy(ef_hbm.at[pl.ds(base, C)], efa_v)
        pltpu.sync_copy(rows_v, s1_sh.at[dst_v], add=True)
        pltpu.sync_copy(efa_v, s2_sh.at[dst_v], add=True)
        return 0
    lax.fori_loop(0, CIT, step, 0)
    plsc.subcore_barrier()

    for k in range(RPW // C):
        ofk = r0 + k * C
        pltpu.sync_copy(s1_sh.at[pl.ds(ofk, C)], rows_v)
        pltpu.sync_copy(rows_v, s1o.at[cid, pl.ds(ofk, C)])
        pltpu.sync_copy(s2_sh.at[pl.ds(ofk, C)], efa_v)
        pltpu.sync_copy(efa_v, s2o.at[cid, pl.ds(ofk, C)])


_sc_agg = pl.kernel(
    _sc_body,
    out_type=(
        jax.ShapeDtypeStruct((NC, H, D), jnp.float32),
        jax.ShapeDtypeStruct((NC, H, DA), jnp.float32),
    ),
    mesh=plsc.VectorSubcoreMesh(core_axis_name="c", subcore_axis_name="s"),
    compiler_params=pltpu.CompilerParams(use_tc_tiling_on_sc=False),
    scratch_types=[
        pltpu.VMEM((C,), jnp.int32),
        pltpu.VMEM((C,), jnp.int32),
        pltpu.VMEM((C, D), jnp.float32),
        pltpu.VMEM((C, DA), jnp.float32),
        pltpu.VMEM_SHARED((HA, D), jnp.float32),
        pltpu.VMEM_SHARED((HA, DA), jnp.float32),
        pltpu.SemaphoreType.DMA,
    ],
)

BT = 640


def _tc_body(nf, s1p, s2p, w1t, w2t, wa1t, wa2t, bm, ba, out):
    s1 = s1p[0]
    s2a = s2p[0]
    s2 = s2a[:, 0:DE]
    cnt = s2a[:, DE:DE + 1]
    msum = (jnp.dot(s1, w1t[...], preferred_element_type=jnp.float32)
            + jnp.dot(s2, w2t[...], preferred_element_type=jnp.float32)
            + cnt * bm[...])
    hn = msum / jnp.maximum(cnt, 1.0)
    h = (jnp.dot(nf[...], wa1t[...], preferred_element_type=jnp.float32)
         + jnp.dot(hn, wa2t[...], preferred_element_type=jnp.float32)
         + ba[...])
    out[...] = jnp.maximum(h, 0.0)


_tc_apply = pl.pallas_call(
    _tc_body,
    grid=(NC, H // BT),
    in_specs=[
        pl.BlockSpec((BT, D), lambda c, j: (c * (H // BT) + j, 0)),
        pl.BlockSpec((1, BT, D), lambda c, j: (c, j, 0)),
        pl.BlockSpec((1, BT, DA), lambda c, j: (c, j, 0)),
        pl.BlockSpec((D, DO), lambda c, j: (0, 0)),
        pl.BlockSpec((DE, DO), lambda c, j: (0, 0)),
        pl.BlockSpec((D, DO), lambda c, j: (0, 0)),
        pl.BlockSpec((DO, DO), lambda c, j: (0, 0)),
        pl.BlockSpec((1, DO), lambda c, j: (0, 0)),
        pl.BlockSpec((1, DO), lambda c, j: (0, 0)),
    ],
    out_specs=pl.BlockSpec((BT, DO), lambda c, j: (c * (H // BT) + j, 0)),
    out_shape=jax.ShapeDtypeStruct((N, DO), jnp.float32),
)


def kernel(nfeats, edge_index, efeats, W_msg, b_msg, W_apply, b_apply):
    zbig = jnp.zeros((C, D), jnp.float32)
    zsmall = jnp.zeros((C, DA), jnp.float32)
    efa = jnp.concatenate(
        [efeats,
         jnp.ones((E, 1), jnp.float32),
         jnp.zeros((E, DA - DE - 1), jnp.float32)], axis=1)
    dst = edge_index[1]
    dst_both = jnp.concatenate([
        jnp.where(dst < H, dst, H),
        jnp.where(dst >= H, dst - H, H),
    ]).astype(jnp.int32)
    s1p, s2p = _sc_agg(nfeats, edge_index[0], dst_both, efa,
                       zbig, zsmall)
    w1t = W_msg[:, :D].T
    w2t = W_msg[:, D:].T
    wa1t = W_apply[:, :D].T
    wa2t = W_apply[:, D:].T
    return _tc_apply(nfeats, s1p, s2p, w1t, w2t, wa1t, wa2t,
                     b_msg.reshape(1, DO), b_apply.reshape(1, DO))


# speedup vs baseline: 2.4270x; 1.0011x over previous
import jax
import jax.numpy as jnp
from jax import lax
from jax.experimental import pallas as pl
from jax.experimental.pallas import tpu as pltpu
from jax.experimental.pallas import tpu_sc as plsc

N = 10000
E = 320000
D = 128
DE = 16
DO = 128
DA = 32

NC = 2
NS = 16
H = 5120
HA = H + 8
EPT = E // NS
C = 80
NB = 2
CIT = EPT // C
RPW = H // NS


def _sc_body(nf_hbm, src_hbm, dst_hbm, ef_hbm, zbig_hbm, zsmall_hbm,
             s1o, s2o,
             src_v, dst_v, rows_v, efa_v,
             s1_sh, s2_sh, sem):
    cid = lax.axis_index("c")
    sid = lax.axis_index("s")
    off = cid * H

    r0 = sid * RPW
    pltpu.sync_copy(zbig_hbm, rows_v.at[0])
    pltpu.sync_copy(zsmall_hbm, efa_v.at[0])
    for k in range(RPW // C):
        pltpu.sync_copy(rows_v.at[0], s1_sh.at[pl.ds(r0 + k * C, C)])
        pltpu.sync_copy(efa_v.at[0], s2_sh.at[pl.ds(r0 + k * C, C)])
    plsc.subcore_barrier()

    e0 = sid * EPT

    @plsc.parallel_loop(0, CIT)
    def step(it):
        b = lax.rem(it, NB)
        base = e0 + it * C
        pltpu.sync_copy(src_hbm.at[pl.ds(base, C)], src_v.at[b])
        pltpu.sync_copy(dst_hbm.at[pl.ds(cid * E + base, C)], dst_v.at[b])
        pltpu.sync_copy(nf_hbm.at[src_v.at[b]], rows_v.at[b])
        pltpu.sync_copy(ef_hbm.at[pl.ds(base, C)], efa_v.at[b])
        pltpu.sync_copy(rows_v.at[b], s1_sh.at[dst_v.at[b]], add=True)
        pltpu.sync_copy(efa_v.at[b], s2_sh.at[dst_v.at[b]], add=True)

    plsc.subcore_barrier()

    for k in range(RPW // C):
        ofk = r0 + k * C
        pltpu.sync_copy(s1_sh.at[pl.ds(ofk, C)], rows_v.at[0])
        pltpu.sync_copy(rows_v.at[0], s1o.at[cid, pl.ds(ofk, C)])
        pltpu.sync_copy(s2_sh.at[pl.ds(ofk, C)], efa_v.at[0])
        pltpu.sync_copy(efa_v.at[0], s2o.at[cid, pl.ds(ofk, C)])


_sc_agg = pl.kernel(
    _sc_body,
    out_type=(
        jax.ShapeDtypeStruct((NC, H, D), jnp.float32),
        jax.ShapeDtypeStruct((NC, H, DA), jnp.float32),
    ),
    mesh=plsc.VectorSubcoreMesh(core_axis_name="c", subcore_axis_name="s"),
    compiler_params=pltpu.CompilerParams(use_tc_tiling_on_sc=False),
    scratch_types=[
        pltpu.VMEM((NB, C), jnp.int32),
        pltpu.VMEM((NB, C), jnp.int32),
        pltpu.VMEM((NB, C, D), jnp.float32),
        pltpu.VMEM((NB, C, DA), jnp.float32),
        pltpu.VMEM_SHARED((HA, D), jnp.float32),
        pltpu.VMEM_SHARED((HA, DA), jnp.float32),
        pltpu.SemaphoreType.DMA,
    ],
)

BT = 640


def _tc_body(nf, s1p, s2p, w1t, w2t, wa1t, wa2t, bm, ba, out):
    s1 = s1p[0]
    s2a = s2p[0]
    s2 = s2a[:, 0:DE]
    cnt = s2a[:, DE:DE + 1]
    msum = (jnp.dot(s1, w1t[...], preferred_element_type=jnp.float32)
            + jnp.dot(s2, w2t[...], preferred_element_type=jnp.float32)
            + cnt * bm[...])
    hn = msum / jnp.maximum(cnt, 1.0)
    h = (jnp.dot(nf[...], wa1t[...], preferred_element_type=jnp.float32)
         + jnp.dot(hn, wa2t[...], preferred_element_type=jnp.float32)
         + ba[...])
    out[...] = jnp.maximum(h, 0.0)


_tc_apply = pl.pallas_call(
    _tc_body,
    grid=(NC, H // BT),
    in_specs=[
        pl.BlockSpec((BT, D), lambda c, j: (c * (H // BT) + j, 0)),
        pl.BlockSpec((1, BT, D), lambda c, j: (c, j, 0)),
        pl.BlockSpec((1, BT, DA), lambda c, j: (c, j, 0)),
        pl.BlockSpec((D, DO), lambda c, j: (0, 0)),
        pl.BlockSpec((DE, DO), lambda c, j: (0, 0)),
        pl.BlockSpec((D, DO), lambda c, j: (0, 0)),
        pl.BlockSpec((DO, DO), lambda c, j: (0, 0)),
        pl.BlockSpec((1, DO), lambda c, j: (0, 0)),
        pl.BlockSpec((1, DO), lambda c, j: (0, 0)),
    ],
    out_specs=pl.BlockSpec((BT, DO), lambda c, j: (c * (H // BT) + j, 0)),
    out_shape=jax.ShapeDtypeStruct((N, DO), jnp.float32),
)


def kernel(nfeats, edge_index, efeats, W_msg, b_msg, W_apply, b_apply):
    zbig = jnp.zeros((C, D), jnp.float32)
    zsmall = jnp.zeros((C, DA), jnp.float32)
    efa = jnp.concatenate(
        [efeats,
         jnp.ones((E, 1), jnp.float32),
         jnp.zeros((E, DA - DE - 1), jnp.float32)], axis=1)
    dst = edge_index[1]
    dst_both = jnp.concatenate([
        jnp.where(dst < H, dst, H),
        jnp.where(dst >= H, dst - H, H),
    ]).astype(jnp.int32)
    s1p, s2p = _sc_agg(nfeats, edge_index[0], dst_both, efa,
                       zbig, zsmall)
    w1t = W_msg[:, :D].T
    w2t = W_msg[:, D:].T
    wa1t = W_apply[:, :D].T
    wa2t = W_apply[:, D:].T
    return _tc_apply(nfeats, s1p, s2p, w1t, w2t, wa1t, wa2t,
                     b_msg.reshape(1, DO), b_apply.reshape(1, DO))
